# baseline (speedup 1.0000x reference)
import functools

import jax
import jax.numpy as jnp
from jax import lax
from jax.experimental import pallas as pl
from jax.experimental.pallas import tpu as pltpu
from jax.experimental.pallas import tpu_sc as plsc

N = 1048576
NC, NS = 2, 16
NW = NC * NS
PTS_W = N // NW
C = 1024
G = C // 16
NCHUNK = PTS_W // C

T = 32768
D0 = 17 ** 3
D1 = 25 ** 3
PRIME1 = 2654435761
PRIME2 = 805459861
RES = (16, 24, 36, 54)
PA_WORDS = D0 * 2 + T * 2
PB_WORDS = D1 * 2 + T * 2


def _corner_weights(wx, wy, wz):
    ux, uy, uz = 1.0 - wx, 1.0 - wy, 1.0 - wz
    wxy = (ux * uy, wx * uy, ux * wy, wx * wy)
    return [wxy[c & 3] * (wz if c & 4 else uz) for c in range(8)]


def _coords(x, y, z, res):
    fres = jnp.float32(res)
    px, py, pz = x * fres, y * fres, z * fres
    cx = px.astype(jnp.int32)
    cy = py.astype(jnp.int32)
    cz = pz.astype(jnp.int32)
    wx = px - cx.astype(jnp.float32)
    wy = py - cy.astype(jnp.float32)
    wz = pz - cz.astype(jnp.float32)
    return cx, cy, cz, wx, wy, wz


def _encode_body(pts_w, nchunk, xs_hbm, ys_hbm, zs_hbm, pa_hbm, pb_hbm,
                 feats_hbm, tab_v, xs_v, ys_v, zs_v, feats_v,
                 is0, is1, os0, os1):
    cid = lax.axis_index("c")
    sid = lax.axis_index("s")
    wid = sid * NC + cid
    in_sems = (is0, is1)
    out_sems = (os0, os1)

    def load_xyz(b, g):
        sl = pl.ds(g * 16, 16)
        return xs_v[b, sl], ys_v[b, sl], zs_v[b, sl]

    def gather_acc(b, idx2_list, wgt, out_base, g):
        acc = [jnp.zeros((16,), jnp.float32) for _ in range(4)]
        himask = jnp.uint32(0xFFFF0000)
        for c in range(8):
            w0 = plsc.bitcast(plsc.load_gather(tab_v, [idx2_list[c]]),
                              jnp.uint32)
            w1 = plsc.bitcast(plsc.load_gather(tab_v, [idx2_list[c] + 1]),
                              jnp.uint32)
            f0 = plsc.bitcast(w0 << 16, jnp.float32)
            f1 = plsc.bitcast(w0 & himask, jnp.float32)
            f2 = plsc.bitcast(w1 << 16, jnp.float32)
            f3 = plsc.bitcast(w1 & himask, jnp.float32)
            wc = wgt[c]
            acc[0] = acc[0] + wc * f0
            acc[1] = acc[1] + wc * f1
            acc[2] = acc[2] + wc * f2
            acc[3] = acc[3] + wc * f3
        sl = pl.ds(g * 16, 16)
        for f in range(4):
            feats_v[b, out_base + f, sl] = acc[f]

    def dense_level(b, res, base_word, out_base, g, x, y, z):
        cx, cy, cz, wx, wy, wz = _coords(x, y, z, res)
        wgt = _corner_weights(wx, wy, wz)
        s1, s2 = res + 1, (res + 1) ** 2
        b2 = (cx + cy * s1 + cz * s2) * 2 + base_word
        idx2 = [b2 + ((c & 1) + ((c >> 1) & 1) * s1 + ((c >> 2) & 1) * s2) * 2
                for c in range(8)]
        gather_acc(b, idx2, wgt, out_base, g)

    def hash_level(b, res, base_word, out_base, g, x, y, z):
        cx, cy, cz, wx, wy, wz = _coords(x, y, z, res)
        wgt = _corner_weights(wx, wy, wz)
        hx0 = plsc.bitcast(cx, jnp.uint32)
        hx1 = hx0 + jnp.uint32(1)
        hy0 = plsc.bitcast(cy, jnp.uint32) * jnp.uint32(PRIME1)
        hy1 = hy0 + jnp.uint32(PRIME1)
        hz0 = plsc.bitcast(cz, jnp.uint32) * jnp.uint32(PRIME2)
        hz1 = hz0 + jnp.uint32(PRIME2)
        hxy = (hx0 ^ hy0, hx1 ^ hy0, hx0 ^ hy1, hx1 ^ hy1)
        idx2 = []
        for c in range(8):
            hz = hz1 if c & 4 else hz0
            idx = plsc.bitcast((hxy[c & 3] ^ hz) & jnp.uint32(T - 1), jnp.int32)
            idx2.append(idx * 2 + base_word)
        gather_acc(b, idx2, wgt, out_base, g)

    def run_pass(tab_src, words, dense_res, dense_row, hash_res, hash_row):
        pltpu.sync_copy(tab_src, tab_v.at[pl.ds(0, words)])
        dense_rows = {16: D0, 24: D1}[dense_res]

        def issue_in(ci, b):
            base = wid * pts_w + ci * C
            pltpu.async_copy(xs_hbm.at[pl.ds(base, C)], xs_v.at[b], in_sems[b])
            pltpu.async_copy(ys_hbm.at[pl.ds(base, C)], ys_v.at[b], in_sems[b])
            pltpu.async_copy(zs_hbm.at[pl.ds(base, C)], zs_v.at[b], in_sems[b])

        def wait_in(b):
            for v in (xs_v, ys_v, zs_v):
                pltpu.make_async_copy(
                    xs_hbm.at[pl.ds(0, C)], v.at[b], in_sems[b]).wait()

        def issue_out(ci, b):
            base = wid * pts_w + ci * C
            pltpu.async_copy(
                feats_v.at[b, pl.ds(0, 4)],
                feats_hbm.at[pl.ds(dense_row, 4), pl.ds(base, C)], out_sems[b])
            pltpu.async_copy(
                feats_v.at[b, pl.ds(4, 4)],
                feats_hbm.at[pl.ds(hash_row, 4), pl.ds(base, C)], out_sems[b])

        def wait_out(b):
            for row in (dense_row, hash_row):
                pltpu.make_async_copy(
                    feats_v.at[b, pl.ds(0, 4)],
                    feats_hbm.at[pl.ds(row, 4), pl.ds(0, C)],
                    out_sems[b]).wait()

        issue_in(0, 0)
        issue_in(1, 1)

        def chunk_pair(cp, _):
            for b in range(2):
                ci = cp * 2 + b
                wait_in(b)

                @pl.when(cp > 0)
                def _():
                    wait_out(b)

                @plsc.parallel_loop(0, G, unroll=4)
                def group(g):
                    x, y, z = load_xyz(b, g)
                    dense_level(b, dense_res, 0, 0, g, x, y, z)
                    hash_level(b, hash_res, dense_rows * 2, 4, g, x, y, z)

                issue_out(ci, b)

                @pl.when(ci + 2 < nchunk)
                def _():
                    issue_in(ci + 2, b)
            return 0

        lax.fori_loop(0, nchunk // 2, chunk_pair, 0)
        wait_out(0)
        wait_out(1)

    run_pass(pa_hbm, PA_WORDS, RES[0], 0, RES[2], 8)
    run_pass(pb_hbm, PB_WORDS, RES[1], 4, RES[3], 12)


def _encode(xs, ys, zs, pa, pb):
    n = xs.shape[0]
    pts_w = n // NW
    return pl.kernel(
        functools.partial(_encode_body, pts_w, pts_w // C),
        out_type=jax.ShapeDtypeStruct((16, n), jnp.float32),
        mesh=plsc.VectorSubcoreMesh(core_axis_name="c", subcore_axis_name="s"),
        compiler_params=pltpu.CompilerParams(needs_layout_passes=False),
        scratch_types=[
            pltpu.VMEM((PB_WORDS,), jnp.float32),
            pltpu.VMEM((2, C), jnp.float32),
            pltpu.VMEM((2, C), jnp.float32),
            pltpu.VMEM((2, C), jnp.float32),
            pltpu.VMEM((2, 8, C), jnp.float32),
            pltpu.SemaphoreType.DMA,
            pltpu.SemaphoreType.DMA,
            pltpu.SemaphoreType.DMA,
            pltpu.SemaphoreType.DMA,
        ],
    )(xs, ys, zs, pa, pb)


BN = 8192


def _mlp_body(f_ref, w1, b1, w2, b2, w3, b3, w4, b4, ox_ref, oy_ref, oz_ref):
    h = f_ref[...]
    for w, b in ((w1, b1), (w2, b2), (w3, b3)):
        z = jnp.dot(w[...], h, preferred_element_type=jnp.float32) + b[...]
        h = jnp.where(z >= 0, z, 0.2 * z)
    z4 = jnp.tanh(
        jnp.dot(w4[...], h, preferred_element_type=jnp.float32) + b4[...])
    ox_ref[...] = z4[0, :]
    oy_ref[...] = z4[1, :]
    oz_ref[...] = z4[2, :]


def _mlp(feats, w1, b1, w2, b2, w3, b3, w4, b4):
    n = feats.shape[1]
    full = lambda a: pl.BlockSpec(a.shape, lambda j: (0, 0))
    return pl.pallas_call(
        _mlp_body,
        grid=(n // BN,),
        in_specs=[pl.BlockSpec((16, BN), lambda j: (0, j)),
                  full(w1), full(b1), full(w2), full(b2),
                  full(w3), full(b3), full(w4), full(b4)],
        out_specs=[pl.BlockSpec((BN,), lambda j: (j,))] * 3,
        out_shape=[jax.ShapeDtypeStruct((n,), jnp.float32)] * 3,
    )(feats, w1, b1, w2, b2, w3, b3, w4, b4)


def _pack_level(tb, l, rows):
    return lax.bitcast_convert_type(
        tb[l, :rows].reshape(rows, 2, 2), jnp.float32).reshape(-1)


def kernel(directions, tables, W1, b1, W2, b2, W3, b3, W4, b4):
    tb = tables.astype(jnp.bfloat16)
    pa = jnp.concatenate([_pack_level(tb, 0, D0), _pack_level(tb, 2, T)])
    pb = jnp.concatenate([_pack_level(tb, 1, D1), _pack_level(tb, 3, T)])
    xs, ys, zs = directions[:, 0], directions[:, 1], directions[:, 2]
    h = N // 4
    outs = []
    for p in range(4):
        sl = slice(p * h, (p + 1) * h)
        feats = _encode(xs[sl], ys[sl], zs[sl], pa, pb)
        ox, oy, oz = _mlp(feats,
                          W1.T, b1.reshape(-1, 1), W2.T, b2.reshape(-1, 1),
                          W3.T, b3.reshape(-1, 1), W4.T, b4.reshape(-1, 1))
        outs.append(jnp.stack([ox, oy, oz], axis=-1))
    return jnp.concatenate(outs, axis=0)


# speedup vs baseline: 1.1327x; 1.1327x over previous
import functools

import jax
import jax.numpy as jnp
from jax import lax
from jax.experimental import pallas as pl
from jax.experimental.pallas import tpu as pltpu
from jax.experimental.pallas import tpu_sc as plsc

N = 1048576
NC, NS = 2, 16
NW = NC * NS
PTS_W = N // NW
C = 1024
G = C // 16
NCHUNK = PTS_W // C

T = 32768
D0 = 17 ** 3
D1 = 25 ** 3
PRIME1 = 2654435761
PRIME2 = 805459861
RES = (16, 24, 36, 54)
PA_WORDS = D0 * 2 + T * 2
PB_WORDS = D1 * 2 + T * 2


def _corner_weights(wx, wy, wz):
    ux, uy, uz = 1.0 - wx, 1.0 - wy, 1.0 - wz
    wxy = (ux * uy, wx * uy, ux * wy, wx * wy)
    return [wxy[c & 3] * (wz if c & 4 else uz) for c in range(8)]


def _coords(x, y, z, res):
    fres = jnp.float32(res)
    px, py, pz = x * fres, y * fres, z * fres
    cx = px.astype(jnp.int32)
    cy = py.astype(jnp.int32)
    cz = pz.astype(jnp.int32)
    wx = px - cx.astype(jnp.float32)
    wy = py - cy.astype(jnp.float32)
    wz = pz - cz.astype(jnp.float32)
    return cx, cy, cz, wx, wy, wz


def _encode_body(pts_w, nchunk, xs_hbm, ys_hbm, zs_hbm, pa_hbm, pb_hbm,
                 feats_hbm, tab_v, xs_v, ys_v, zs_v, feats_v,
                 is0, is1, os0, os1):
    cid = lax.axis_index("c")
    sid = lax.axis_index("s")
    wid = sid * NC + cid
    in_sems = (is0, is1)
    out_sems = (os0, os1)

    def load_xyz(b, g):
        sl = pl.ds(g * 16, 16)
        return xs_v[b, sl], ys_v[b, sl], zs_v[b, sl]

    def gather_acc(b, idx2_list, wgt, out_base, g):
        acc = [jnp.zeros((16,), jnp.float32) for _ in range(4)]
        himask = jnp.uint32(0xFFFF0000)
        for c in range(8):
            w0 = plsc.bitcast(plsc.load_gather(tab_v, [idx2_list[c]]),
                              jnp.uint32)
            w1 = plsc.bitcast(plsc.load_gather(tab_v, [idx2_list[c] + 1]),
                              jnp.uint32)
            f0 = plsc.bitcast(w0 << 16, jnp.float32)
            f1 = plsc.bitcast(w0 & himask, jnp.float32)
            f2 = plsc.bitcast(w1 << 16, jnp.float32)
            f3 = plsc.bitcast(w1 & himask, jnp.float32)
            wc = wgt[c]
            acc[0] = acc[0] + wc * f0
            acc[1] = acc[1] + wc * f1
            acc[2] = acc[2] + wc * f2
            acc[3] = acc[3] + wc * f3
        sl = pl.ds(g * 16, 16)
        for f in range(4):
            feats_v[b, out_base + f, sl] = acc[f]

    def dense_level(b, res, base_word, out_base, g, x, y, z):
        cx, cy, cz, wx, wy, wz = _coords(x, y, z, res)
        wgt = _corner_weights(wx, wy, wz)
        s1, s2 = res + 1, (res + 1) ** 2
        b2 = (cx + cy * s1 + cz * s2) * 2 + base_word
        idx2 = [b2 + ((c & 1) + ((c >> 1) & 1) * s1 + ((c >> 2) & 1) * s2) * 2
                for c in range(8)]
        gather_acc(b, idx2, wgt, out_base, g)

    def hash_level(b, res, base_word, out_base, g, x, y, z):
        cx, cy, cz, wx, wy, wz = _coords(x, y, z, res)
        wgt = _corner_weights(wx, wy, wz)
        hx0 = plsc.bitcast(cx * 2, jnp.uint32)
        hx1 = hx0 + jnp.uint32(2)
        hy0 = plsc.bitcast(cy, jnp.uint32) * jnp.uint32((PRIME1 * 2) & 0xFFFFFFFF)
        hy1 = hy0 + jnp.uint32((PRIME1 * 2) & 0xFFFFFFFF)
        hz0 = plsc.bitcast(cz, jnp.uint32) * jnp.uint32((PRIME2 * 2) & 0xFFFFFFFF)
        hz1 = hz0 + jnp.uint32((PRIME2 * 2) & 0xFFFFFFFF)
        hxy = (hx0 ^ hy0, hx1 ^ hy0, hx0 ^ hy1, hx1 ^ hy1)
        idx2 = []
        for c in range(8):
            hz = hz1 if c & 4 else hz0
            i2 = plsc.bitcast((hxy[c & 3] ^ hz) & jnp.uint32(2 * T - 2),
                              jnp.int32)
            idx2.append(i2 + base_word if base_word else i2)
        gather_acc(b, idx2, wgt, out_base, g)

    def run_pass(tab_src, words, dense_res, dense_row, hash_res, hash_row):
        pltpu.sync_copy(tab_src, tab_v.at[pl.ds(0, words)])
        dense_rows = {16: D0, 24: D1}[dense_res]

        def issue_in(ci, b):
            base = wid * pts_w + ci * C
            pltpu.async_copy(xs_hbm.at[pl.ds(base, C)], xs_v.at[b], in_sems[b])
            pltpu.async_copy(ys_hbm.at[pl.ds(base, C)], ys_v.at[b], in_sems[b])
            pltpu.async_copy(zs_hbm.at[pl.ds(base, C)], zs_v.at[b], in_sems[b])

        def wait_in(b):
            for v in (xs_v, ys_v, zs_v):
                pltpu.make_async_copy(
                    xs_hbm.at[pl.ds(0, C)], v.at[b], in_sems[b]).wait()

        def issue_out(ci, b):
            base = wid * pts_w + ci * C
            pltpu.async_copy(
                feats_v.at[b, pl.ds(0, 4)],
                feats_hbm.at[pl.ds(dense_row, 4), pl.ds(base, C)], out_sems[b])
            pltpu.async_copy(
                feats_v.at[b, pl.ds(4, 4)],
                feats_hbm.at[pl.ds(hash_row, 4), pl.ds(base, C)], out_sems[b])

        def wait_out(b):
            for row in (dense_row, hash_row):
                pltpu.make_async_copy(
                    feats_v.at[b, pl.ds(0, 4)],
                    feats_hbm.at[pl.ds(row, 4), pl.ds(0, C)],
                    out_sems[b]).wait()

        issue_in(0, 0)
        issue_in(1, 1)

        def chunk_pair(cp, _):
            for b in range(2):
                ci = cp * 2 + b
                wait_in(b)

                @pl.when(cp > 0)
                def _():
                    wait_out(b)

                @plsc.parallel_loop(0, G, unroll=4)
                def group(g):
                    x, y, z = load_xyz(b, g)
                    dense_level(b, dense_res, 0, 0, g, x, y, z)
                    hash_level(b, hash_res, dense_rows * 2, 4, g, x, y, z)

                issue_out(ci, b)

                @pl.when(ci + 2 < nchunk)
                def _():
                    issue_in(ci + 2, b)
            return 0

        lax.fori_loop(0, nchunk // 2, chunk_pair, 0)
        wait_out(0)
        wait_out(1)

    run_pass(pa_hbm, PA_WORDS, RES[0], 0, RES[2], 8)
    run_pass(pb_hbm, PB_WORDS, RES[1], 4, RES[3], 12)


def _encode(xs, ys, zs, pa, pb):
    n = xs.shape[0]
    pts_w = n // NW
    return pl.kernel(
        functools.partial(_encode_body, pts_w, pts_w // C),
        out_type=jax.ShapeDtypeStruct((16, n), jnp.float32),
        mesh=plsc.VectorSubcoreMesh(core_axis_name="c", subcore_axis_name="s"),
        compiler_params=pltpu.CompilerParams(needs_layout_passes=False),
        scratch_types=[
            pltpu.VMEM((PB_WORDS,), jnp.float32),
            pltpu.VMEM((2, C), jnp.float32),
            pltpu.VMEM((2, C), jnp.float32),
            pltpu.VMEM((2, C), jnp.float32),
            pltpu.VMEM((2, 8, C), jnp.float32),
            pltpu.SemaphoreType.DMA,
            pltpu.SemaphoreType.DMA,
            pltpu.SemaphoreType.DMA,
            pltpu.SemaphoreType.DMA,
        ],
    )(xs, ys, zs, pa, pb)


BN = 8192


def _mlp_body(f_ref, w1, b1, w2, b2, w3, b3, w4, b4, ox_ref, oy_ref, oz_ref):
    h = f_ref[...]
    for w, b in ((w1, b1), (w2, b2), (w3, b3)):
        z = jnp.dot(w[...], h, preferred_element_type=jnp.float32) + b[...]
        h = jnp.where(z >= 0, z, 0.2 * z)
    z4 = jnp.tanh(
        jnp.dot(w4[...], h, preferred_element_type=jnp.float32) + b4[...])
    ox_ref[...] = z4[0, :]
    oy_ref[...] = z4[1, :]
    oz_ref[...] = z4[2, :]


def _mlp(feats, w1, b1, w2, b2, w3, b3, w4, b4):
    n = feats.shape[1]
    full = lambda a: pl.BlockSpec(a.shape, lambda j: (0, 0))
    return pl.pallas_call(
        _mlp_body,
        grid=(n // BN,),
        in_specs=[pl.BlockSpec((16, BN), lambda j: (0, j)),
                  full(w1), full(b1), full(w2), full(b2),
                  full(w3), full(b3), full(w4), full(b4)],
        out_specs=[pl.BlockSpec((BN,), lambda j: (j,))] * 3,
        out_shape=[jax.ShapeDtypeStruct((n,), jnp.float32)] * 3,
    )(feats, w1, b1, w2, b2, w3, b3, w4, b4)


def _pack_level(tb, l, rows):
    return lax.bitcast_convert_type(
        tb[l, :rows].reshape(rows, 2, 2), jnp.float32).reshape(-1)


def kernel(directions, tables, W1, b1, W2, b2, W3, b3, W4, b4):
    tb = tables.astype(jnp.bfloat16)
    pa = jnp.concatenate([_pack_level(tb, 0, D0), _pack_level(tb, 2, T)])
    pb = jnp.concatenate([_pack_level(tb, 1, D1), _pack_level(tb, 3, T)])
    xs, ys, zs = directions[:, 0], directions[:, 1], directions[:, 2]
    h = N // 2
    outs = []
    for p in range(2):
        sl = slice(p * h, (p + 1) * h)
        feats = _encode(xs[sl], ys[sl], zs[sl], pa, pb)
        ox, oy, oz = _mlp(feats,
                          W1.T, b1.reshape(-1, 1), W2.T, b2.reshape(-1, 1),
                          W3.T, b3.reshape(-1, 1), W4.T, b4.reshape(-1, 1))
        outs.append(jnp.stack([ox, oy, oz], axis=-1))
    return jnp.concatenate(outs, axis=0)
